# baseline (speedup 1.0000x reference)
import functools

import jax
import jax.numpy as jnp
from jax import lax
from jax.experimental import pallas as pl
from jax.experimental.pallas import tpu as pltpu

_BF = jnp.bfloat16
_F32 = jnp.float32
_VMEM = 64 * 1024 * 1024

_INC = {
    "i3a": (192, 64, 96, 128, 16, 32, 32, 2),
    "i3b": (256, 128, 128, 192, 32, 96, 64, 2),
    "i4a": (480, 192, 96, 208, 16, 48, 64, 4),
    "i4b": (512, 160, 112, 224, 24, 64, 64, 4),
    "i4c": (512, 128, 128, 256, 24, 64, 64, 4),
    "i4d": (512, 112, 144, 288, 32, 64, 64, 4),
    "i4e": (528, 256, 160, 320, 32, 128, 128, 4),
    "i5a": (832, 256, 160, 320, 32, 128, 128, 8),
    "i5b": (832, 384, 192, 384, 48, 128, 128, 8),
}


def _cparams():
    return pltpu.CompilerParams(dimension_semantics=("parallel",),
                                vmem_limit_bytes=_VMEM)


def _full_spec(a):
    n = a.ndim
    return pl.BlockSpec(a.shape, lambda i, _n=n: (0,) * _n)


def _pad_hw(v, top, bottom, left, right):
    nb, H, W, C = v.shape
    dt = v.dtype
    if left or right:
        pieces = []
        if left:
            pieces.append(jnp.zeros((nb, H, left, C), dt))
        pieces.append(v)
        if right:
            pieces.append(jnp.zeros((nb, H, right, C), dt))
        v = jnp.concatenate(pieces, axis=2)
    if top or bottom:
        W2 = v.shape[2]
        pieces = []
        if top:
            pieces.append(jnp.zeros((nb, top, W2, C), dt))
        pieces.append(v)
        if bottom:
            pieces.append(jnp.zeros((nb, bottom, W2, C), dt))
        v = jnp.concatenate(pieces, axis=1)
    return v


def _maxpool3s1(v):
    nb, H, W, C = v.shape
    p = _pad_hw(v, 1, 1, 1, 1)
    out = None
    for di in range(3):
        for dj in range(3):
            s = p[:, di:di + H, dj:dj + W, :]
            out = s if out is None else jnp.maximum(out, s)
    return out


def _maxpool3s2(v):
    nb, H, W, C = v.shape
    Ho, Wo = H // 2, W // 2
    p = _pad_hw(v, 0, 2, 0, 0)
    r = p.reshape(nb, Ho + 1, 2, W, C)
    a = jnp.maximum(jnp.maximum(r[:, :Ho, 0], r[:, :Ho, 1]), r[:, 1:Ho + 1, 0])
    p2 = _pad_hw(a, 0, 0, 0, 2)
    c = p2.reshape(nb, Ho, Wo + 1, 2, C)
    return jnp.maximum(jnp.maximum(c[:, :, :Wo, 0], c[:, :, :Wo, 1]),
                       c[:, :, 1:Wo + 1, 0])


def _maxpool2s2(v):
    nb, H, W, C = v.shape
    Ho, Wo = H // 2, W // 2
    r = v.reshape(nb, Ho, 2, W, C)
    a = jnp.maximum(r[:, :, 0], r[:, :, 1])
    c = a.reshape(nb, Ho, Wo, 2, C)
    return jnp.maximum(c[:, :, :, 0], c[:, :, :, 1])


def _conv3x3(vpad, w3_ref, H, W):
    nb = vpad.shape[0]
    Cr = vpad.shape[-1]
    acc = None
    for di in range(3):
        cat = jnp.concatenate(
            [vpad[:, di:di + H, dj:dj + W, :] for dj in range(3)], axis=-1)
        d = jnp.dot(cat.reshape(nb * H * W, 3 * Cr), w3_ref[di],
                    preferred_element_type=_F32)
        acc = d if acc is None else acc + d
    return acc


def _stem1_kernel(pat_ref, w_ref, b_ref, o_ref):
    a = jnp.dot(pat_ref[0], w_ref[...], preferred_element_type=_F32)
    a = jnp.maximum(a + b_ref[...], 0.0)
    v = a.astype(_BF).reshape(1, 112, 112, 64)
    o_ref[...] = _maxpool3s2(v)


def _stem2_kernel(x_ref, w2_ref, b2_ref, w3_ref, b3_ref, o_ref):
    x = x_ref[...]
    y2 = jnp.dot(x.reshape(3136, 64), w2_ref[...], preferred_element_type=_F32)
    y2 = jnp.maximum(y2 + b2_ref[...], 0.0).astype(_BF).reshape(1, 56, 56, 64)
    yp = _pad_hw(y2, 1, 1, 1, 1)
    y3 = _conv3x3(yp, w3_ref, 56, 56)
    y3 = jnp.maximum(y3 + b3_ref[...], 0.0).astype(_BF).reshape(1, 56, 56, 192)
    o_ref[...] = _maxpool3s2(y3)


def _inc_kernel(dims, post, x_ref, w1m_ref, b1m_ref, w1p_ref, b1p_ref,
                w3_ref, b3_ref, o_ref):
    c1, c3, c5, cp, Cr = dims
    nb, H, W, C = x_ref.shape
    M = nb * H * W
    x = x_ref[...]
    pooled = _maxpool3s1(x)
    ymain = jnp.dot(x.reshape(M, C), w1m_ref[...], preferred_element_type=_F32)
    ymain = jnp.maximum(ymain + b1m_ref[...], 0.0)
    b4 = jnp.dot(pooled.reshape(M, C), w1p_ref[...],
                 preferred_element_type=_F32)
    b4 = jnp.maximum(b4 + b1p_ref[...], 0.0)
    red = ymain[:, c1:].astype(_BF).reshape(nb, H, W, Cr)
    y3 = _conv3x3(_pad_hw(red, 1, 1, 1, 1), w3_ref, H, W)
    y3 = jnp.maximum(y3 + b3_ref[...], 0.0)
    parts = ((0, c1, ymain[:, :c1]), (c1, c3 + c5, y3),
             (c1 + c3 + c5, cp, b4))
    if post == "mean":
        for col, wdt, v in parts:
            o_ref[:, col:col + wdt] = jnp.mean(
                v.reshape(nb, H * W, wdt), axis=1)
    else:
        for col, wdt, v in parts:
            vb = v.astype(_BF).reshape(nb, H, W, wdt)
            if post == "pool3":
                vb = _maxpool3s2(vb)
            elif post == "pool2":
                vb = _maxpool2s2(vb)
            o_ref[:, :, :, col:col + wdt] = vb


def _head_kernel(f_ref, fcw_ref, fcb_ref, hw_ref, hb_ref, o_ref):
    t = jnp.dot(f_ref[...].astype(_BF), fcw_ref[...],
                preferred_element_type=_F32)
    t = jnp.maximum(t + fcb_ref[...], 0.0)
    logits = jnp.dot(t.astype(_BF), hw_ref[...],
                     preferred_element_type=_F32) + hb_ref[...]
    col = lax.broadcasted_iota(jnp.int32, logits.shape, 1)
    mask = col < 10
    logits = jnp.where(mask, logits, -jnp.inf)
    m = jnp.max(logits, axis=-1, keepdims=True)
    e = jnp.where(mask, jnp.exp(logits - m), 0.0)
    o_ref[...] = e / jnp.sum(e, axis=-1, keepdims=True)


def _rearrange_w3(w3, Cr, N):
    return (w3.reshape(Cr, 3, 3, N).transpose(1, 2, 0, 3)
            .reshape(3, 3 * Cr, N).astype(_BF))


def _inception(x, name, w1, b1, w3, b3, post):
    cin, c1, c3r, c3, c5r, c5, cp, nb = _INC[name]
    nmain = c1 + c3r + c5r
    Cr = c3r + c5r
    N, H, W, C = x.shape
    w1m = w1[:cin, :nmain].astype(_BF)
    w1p = w1[cin:, nmain:].astype(_BF)
    b1m = b1[:, :nmain]
    b1p = b1[:, nmain:]
    w3r = _rearrange_w3(w3, Cr, c3 + c5)
    cout = c1 + c3 + c5 + cp
    if post == "mean":
        out_shape = jax.ShapeDtypeStruct((N, cout), _F32)
        out_spec = pl.BlockSpec((nb, cout), lambda i: (i, 0))
    elif post in ("pool3", "pool2"):
        out_shape = jax.ShapeDtypeStruct((N, H // 2, W // 2, cout), _BF)
        out_spec = pl.BlockSpec((nb, H // 2, W // 2, cout),
                                lambda i: (i, 0, 0, 0))
    else:
        out_shape = jax.ShapeDtypeStruct((N, H, W, cout), _BF)
        out_spec = pl.BlockSpec((nb, H, W, cout), lambda i: (i, 0, 0, 0))
    kern = functools.partial(_inc_kernel, (c1, c3, c5, cp, Cr), post)
    return pl.pallas_call(
        kern,
        grid=(N // nb,),
        in_specs=[pl.BlockSpec((nb, H, W, C), lambda i: (i, 0, 0, 0)),
                  _full_spec(w1m), _full_spec(b1m), _full_spec(w1p),
                  _full_spec(b1p), _full_spec(w3r), _full_spec(b3)],
        out_specs=out_spec,
        out_shape=out_shape,
        compiler_params=_cparams(),
    )(x, w1m, b1m, w1p, b1p, w3r, b3)


def kernel(x, conv1_wmat, conv1_bias, conv2_wmat, conv2_bias, conv3_wmat,
           conv3_bias, i3a_w1, i3a_b1, i3a_w3, i3a_b3, i3b_w1, i3b_b1,
           i3b_w3, i3b_b3, i4a_w1, i4a_b1, i4a_w3, i4a_b3, i4b_w1, i4b_b1,
           i4b_w3, i4b_b3, i4c_w1, i4c_b1, i4c_w3, i4c_b3, i4d_w1, i4d_b1,
           i4d_w3, i4d_b3, i4e_w1, i4e_b1, i4e_w3, i4e_b3, i5a_w1, i5a_b1,
           i5a_w3, i5a_b3, i5b_w1, i5b_b1, i5b_w3, i5b_b3, fc_w, fc_b,
           head_w, head_b):
    N = x.shape[0]
    scale = jnp.array([0.229 / 0.5, 0.224 / 0.5, 0.225 / 0.5], _F32)
    shift = jnp.array([(0.485 - 0.5) / 0.5, (0.456 - 0.5) / 0.5,
                       (0.406 - 0.5) / 0.5], _F32)
    xt = (x * scale.reshape(1, 3, 1, 1) + shift.reshape(1, 3, 1, 1))
    xh = jnp.transpose(xt.astype(_BF), (0, 2, 3, 1))
    xp = jnp.pad(xh, ((0, 0), (3, 3), (3, 3), (0, 0)))
    pat = jnp.concatenate(
        [xp[:, i:i + 223:2, j:j + 223:2, :]
         for i in range(7) for j in range(7)], axis=-1).reshape(N, 12544, 147)

    w1 = (conv1_wmat.reshape(3, 7, 7, 64).transpose(1, 2, 0, 3)
          .reshape(147, 64).astype(_BF))
    s1 = pl.pallas_call(
        _stem1_kernel,
        grid=(N,),
        in_specs=[pl.BlockSpec((1, 12544, 147), lambda i: (i, 0, 0)),
                  _full_spec(w1), _full_spec(conv1_bias)],
        out_specs=pl.BlockSpec((1, 56, 56, 64), lambda i: (i, 0, 0, 0)),
        out_shape=jax.ShapeDtypeStruct((N, 56, 56, 64), _BF),
        compiler_params=_cparams(),
    )(pat, w1, conv1_bias)

    w2 = conv2_wmat.astype(_BF)
    w3s = _rearrange_w3(conv3_wmat, 64, 192)
    s2 = pl.pallas_call(
        _stem2_kernel,
        grid=(N,),
        in_specs=[pl.BlockSpec((1, 56, 56, 64), lambda i: (i, 0, 0, 0)),
                  _full_spec(w2), _full_spec(conv2_bias),
                  _full_spec(w3s), _full_spec(conv3_bias)],
        out_specs=pl.BlockSpec((1, 28, 28, 192), lambda i: (i, 0, 0, 0)),
        out_shape=jax.ShapeDtypeStruct((N, 28, 28, 192), _BF),
        compiler_params=_cparams(),
    )(s1, w2, conv2_bias, w3s, conv3_bias)

    v = _inception(s2, "i3a", i3a_w1, i3a_b1, i3a_w3, i3a_b3, "none")
    v = _inception(v, "i3b", i3b_w1, i3b_b1, i3b_w3, i3b_b3, "pool3")
    v = _inception(v, "i4a", i4a_w1, i4a_b1, i4a_w3, i4a_b3, "none")
    v = _inception(v, "i4b", i4b_w1, i4b_b1, i4b_w3, i4b_b3, "none")
    v = _inception(v, "i4c", i4c_w1, i4c_b1, i4c_w3, i4c_b3, "none")
    v = _inception(v, "i4d", i4d_w1, i4d_b1, i4d_w3, i4d_b3, "none")
    v = _inception(v, "i4e", i4e_w1, i4e_b1, i4e_w3, i4e_b3, "pool2")
    v = _inception(v, "i5a", i5a_w1, i5a_b1, i5a_w3, i5a_b3, "none")
    feats = _inception(v, "i5b", i5b_w1, i5b_b1, i5b_w3, i5b_b3, "mean")

    out = pl.pallas_call(
        _head_kernel,
        out_shape=jax.ShapeDtypeStruct((N, head_w.shape[1]), _F32),
    )(feats, fc_w.astype(_BF), fc_b, head_w.astype(_BF), head_b)
    return out[:, :10]


# speedup vs baseline: 1.6131x; 1.6131x over previous
import functools

import numpy as np

import jax
import jax.numpy as jnp
from jax import lax
from jax.experimental import pallas as pl
from jax.experimental.pallas import tpu as pltpu

_BF = jnp.bfloat16
_F32 = jnp.float32
_VMEM = 64 * 1024 * 1024

_INC = {
    "i3a": (192, 64, 96, 128, 16, 32, 32, 2),
    "i3b": (256, 128, 128, 192, 32, 96, 64, 2),
    "i4a": (480, 192, 96, 208, 16, 48, 64, 4),
    "i4b": (512, 160, 112, 224, 24, 64, 64, 4),
    "i4c": (512, 128, 128, 256, 24, 64, 64, 4),
    "i4d": (512, 112, 144, 288, 32, 64, 64, 4),
    "i4e": (528, 256, 160, 320, 32, 128, 128, 4),
    "i5a": (832, 256, 160, 320, 32, 128, 128, 8),
    "i5b": (832, 384, 192, 384, 48, 128, 128, 8),
}


def _cparams():
    return pltpu.CompilerParams(dimension_semantics=("parallel",),
                                vmem_limit_bytes=_VMEM)


def _full_spec(a):
    n = a.ndim
    return pl.BlockSpec(a.shape, lambda i, _n=n: (0,) * _n)


def _pad_hw(v, top, bottom, left, right):
    nb, H, W, C = v.shape
    dt = v.dtype
    if left or right:
        pieces = []
        if left:
            pieces.append(jnp.zeros((nb, H, left, C), dt))
        pieces.append(v)
        if right:
            pieces.append(jnp.zeros((nb, H, right, C), dt))
        v = jnp.concatenate(pieces, axis=2)
    if top or bottom:
        W2 = v.shape[2]
        pieces = []
        if top:
            pieces.append(jnp.zeros((nb, top, W2, C), dt))
        pieces.append(v)
        if bottom:
            pieces.append(jnp.zeros((nb, bottom, W2, C), dt))
        v = jnp.concatenate(pieces, axis=1)
    return v


def _maxpool3s1(v):
    nb, H, W, C = v.shape
    p = _pad_hw(v, 1, 1, 1, 1)
    out = None
    for di in range(3):
        for dj in range(3):
            s = p[:, di:di + H, dj:dj + W, :]
            out = s if out is None else jnp.maximum(out, s)
    return out


def _maxpool3s2(v):
    nb, H, W, C = v.shape
    Ho, Wo = H // 2, W // 2
    p = _pad_hw(v, 0, 2, 0, 0)
    r = p.reshape(nb, Ho + 1, 2, W, C)
    a = jnp.maximum(jnp.maximum(r[:, :Ho, 0], r[:, :Ho, 1]), r[:, 1:Ho + 1, 0])
    p2 = _pad_hw(a, 0, 0, 0, 2)
    c = p2.reshape(nb, Ho, Wo + 1, 2, C)
    return jnp.maximum(jnp.maximum(c[:, :, :Wo, 0], c[:, :, :Wo, 1]),
                       c[:, :, 1:Wo + 1, 0])


def _maxpool2s2(v):
    nb, H, W, C = v.shape
    Ho, Wo = H // 2, W // 2
    r = v.reshape(nb, Ho, 2, W, C)
    a = jnp.maximum(r[:, :, 0], r[:, :, 1])
    c = a.reshape(nb, Ho, Wo, 2, C)
    return jnp.maximum(c[:, :, :, 0], c[:, :, :, 1])


def _conv3x3(vpad, w3_ref, H, W):
    nb = vpad.shape[0]
    Cr = vpad.shape[-1]
    acc = None
    for di in range(3):
        cat = jnp.concatenate(
            [vpad[:, di:di + H, dj:dj + W, :] for dj in range(3)], axis=-1)
        d = jnp.dot(cat.reshape(nb * H * W, 3 * Cr), w3_ref[di],
                    preferred_element_type=_F32)
        acc = d if acc is None else acc + d
    return acc


def _stem1_kernel(x_ref, w_ref, b_ref, o_ref):
    acc = None
    for ra in range(4):
        d = jnp.dot(x_ref[0, ra:ra + 112].reshape(12544, 48), w_ref[ra],
                    preferred_element_type=_F32)
        acc = d if acc is None else acc + d
    a = jnp.maximum(acc + b_ref[...], 0.0)
    v = a.astype(_BF).reshape(1, 112, 112, 64)
    o_ref[...] = _maxpool3s2(v)


def _stem2_kernel(x_ref, w2_ref, b2_ref, w3_ref, b3_ref, o_ref):
    x = x_ref[...]
    y2 = jnp.dot(x.reshape(3136, 64), w2_ref[...], preferred_element_type=_F32)
    y2 = jnp.maximum(y2 + b2_ref[...], 0.0).astype(_BF).reshape(1, 56, 56, 64)
    yp = _pad_hw(y2, 1, 1, 1, 1)
    y3 = _conv3x3(yp, w3_ref, 56, 56)
    y3 = jnp.maximum(y3 + b3_ref[...], 0.0).astype(_BF).reshape(1, 56, 56, 192)
    o_ref[...] = _maxpool3s2(y3)


def _inc_kernel(dims, post, x_ref, w1m_ref, b1m_ref, w1p_ref, b1p_ref,
                w3_ref, b3_ref, o_ref):
    c1, c3, c5, cp, Cr = dims
    nb, H, W, C = x_ref.shape
    M = nb * H * W
    x = x_ref[...]
    pooled = _maxpool3s1(x)
    ymain = jnp.dot(x.reshape(M, C), w1m_ref[...], preferred_element_type=_F32)
    ymain = jnp.maximum(ymain + b1m_ref[...], 0.0)
    b4 = jnp.dot(pooled.reshape(M, C), w1p_ref[...],
                 preferred_element_type=_F32)
    b4 = jnp.maximum(b4 + b1p_ref[...], 0.0)
    red = ymain[:, c1:].astype(_BF).reshape(nb, H, W, Cr)
    y3 = _conv3x3(_pad_hw(red, 1, 1, 1, 1), w3_ref, H, W)
    y3 = jnp.maximum(y3 + b3_ref[...], 0.0)
    parts = ((0, c1, ymain[:, :c1]), (c1, c3 + c5, y3),
             (c1 + c3 + c5, cp, b4))
    if post == "mean":
        for col, wdt, v in parts:
            o_ref[:, col:col + wdt] = jnp.mean(
                v.reshape(nb, H * W, wdt), axis=1)
    else:
        for col, wdt, v in parts:
            vb = v.astype(_BF).reshape(nb, H, W, wdt)
            if post == "pool3":
                vb = _maxpool3s2(vb)
            elif post == "pool2":
                vb = _maxpool2s2(vb)
            o_ref[:, :, :, col:col + wdt] = vb


def _head_kernel(f_ref, fcw_ref, fcb_ref, hw_ref, hb_ref, o_ref):
    t = jnp.dot(f_ref[...].astype(_BF), fcw_ref[...],
                preferred_element_type=_F32)
    t = jnp.maximum(t + fcb_ref[...], 0.0)
    logits = jnp.dot(t.astype(_BF), hw_ref[...],
                     preferred_element_type=_F32) + hb_ref[...]
    col = lax.broadcasted_iota(jnp.int32, logits.shape, 1)
    mask = col < 10
    logits = jnp.where(mask, logits, -jnp.inf)
    m = jnp.max(logits, axis=-1, keepdims=True)
    e = jnp.where(mask, jnp.exp(logits - m), 0.0)
    o_ref[...] = e / jnp.sum(e, axis=-1, keepdims=True)


def _rearrange_w3(w3, Cr, N):
    return (w3.reshape(Cr, 3, 3, N).transpose(1, 2, 0, 3)
            .reshape(3, 3 * Cr, N).astype(_BF))


def _inception(x, name, w1, b1, w3, b3, post):
    cin, c1, c3r, c3, c5r, c5, cp, nb = _INC[name]
    nmain = c1 + c3r + c5r
    Cr = c3r + c5r
    N, H, W, C = x.shape
    w1m = w1[:cin, :nmain].astype(_BF)
    w1p = w1[cin:, nmain:].astype(_BF)
    b1m = b1[:, :nmain]
    b1p = b1[:, nmain:]
    w3r = _rearrange_w3(w3, Cr, c3 + c5)
    cout = c1 + c3 + c5 + cp
    if post == "mean":
        out_shape = jax.ShapeDtypeStruct((N, cout), _F32)
        out_spec = pl.BlockSpec((nb, cout), lambda i: (i, 0))
    elif post in ("pool3", "pool2"):
        out_shape = jax.ShapeDtypeStruct((N, H // 2, W // 2, cout), _BF)
        out_spec = pl.BlockSpec((nb, H // 2, W // 2, cout),
                                lambda i: (i, 0, 0, 0))
    else:
        out_shape = jax.ShapeDtypeStruct((N, H, W, cout), _BF)
        out_spec = pl.BlockSpec((nb, H, W, cout), lambda i: (i, 0, 0, 0))
    kern = functools.partial(_inc_kernel, (c1, c3, c5, cp, Cr), post)
    return pl.pallas_call(
        kern,
        grid=(N // nb,),
        in_specs=[pl.BlockSpec((nb, H, W, C), lambda i: (i, 0, 0, 0)),
                  _full_spec(w1m), _full_spec(b1m), _full_spec(w1p),
                  _full_spec(b1p), _full_spec(w3r), _full_spec(b3)],
        out_specs=out_spec,
        out_shape=out_shape,
        compiler_params=_cparams(),
    )(x, w1m, b1m, w1p, b1p, w3r, b3)


def kernel(x, conv1_wmat, conv1_bias, conv2_wmat, conv2_bias, conv3_wmat,
           conv3_bias, i3a_w1, i3a_b1, i3a_w3, i3a_b3, i3b_w1, i3b_b1,
           i3b_w3, i3b_b3, i4a_w1, i4a_b1, i4a_w3, i4a_b3, i4b_w1, i4b_b1,
           i4b_w3, i4b_b3, i4c_w1, i4c_b1, i4c_w3, i4c_b3, i4d_w1, i4d_b1,
           i4d_w3, i4d_b3, i4e_w1, i4e_b1, i4e_w3, i4e_b3, i5a_w1, i5a_b1,
           i5a_w3, i5a_b3, i5b_w1, i5b_b1, i5b_w3, i5b_b3, fc_w, fc_b,
           head_w, head_b):
    N = x.shape[0]
    scale = jnp.array([0.229 / 0.5, 0.224 / 0.5, 0.225 / 0.5], _F32)
    shift = jnp.array([(0.485 - 0.5) / 0.5, (0.456 - 0.5) / 0.5,
                       (0.406 - 0.5) / 0.5], _F32)
    xt = (x * scale.reshape(1, 3, 1, 1) + shift.reshape(1, 3, 1, 1))
    x12 = (xt.reshape(N, 3, 112, 2, 112, 2).transpose(0, 2, 4, 3, 5, 1)
           .reshape(N, 112, 112, 12).astype(_BF))
    xpp = jnp.pad(x12, ((0, 0), (2, 1), (2, 1), (0, 0)))
    xcat = jnp.concatenate(
        [xpp[:, :, rb:rb + 112, :] for rb in range(4)], axis=-1)

    idx = np.full((4, 4, 2, 2, 3), 147, np.int32)
    for ra in range(4):
        for rb in range(4):
            for p in range(2):
                for q in range(2):
                    i, j = 2 * ra + p - 1, 2 * rb + q - 1
                    if 0 <= i < 7 and 0 <= j < 7:
                        for c in range(3):
                            idx[ra, rb, p, q, c] = c * 49 + i * 7 + j
    w_ext = jnp.concatenate([conv1_wmat, jnp.zeros((1, 64), _F32)], axis=0)
    w1 = jnp.take(w_ext, jnp.asarray(idx.reshape(4, 48)), axis=0).astype(_BF)

    s1 = pl.pallas_call(
        _stem1_kernel,
        grid=(N,),
        in_specs=[pl.BlockSpec((1, 115, 112, 48), lambda i: (i, 0, 0, 0)),
                  _full_spec(w1), _full_spec(conv1_bias)],
        out_specs=pl.BlockSpec((1, 56, 56, 64), lambda i: (i, 0, 0, 0)),
        out_shape=jax.ShapeDtypeStruct((N, 56, 56, 64), _BF),
        compiler_params=_cparams(),
    )(xcat, w1, conv1_bias)

    w2 = conv2_wmat.astype(_BF)
    w3s = _rearrange_w3(conv3_wmat, 64, 192)
    s2 = pl.pallas_call(
        _stem2_kernel,
        grid=(N,),
        in_specs=[pl.BlockSpec((1, 56, 56, 64), lambda i: (i, 0, 0, 0)),
                  _full_spec(w2), _full_spec(conv2_bias),
                  _full_spec(w3s), _full_spec(conv3_bias)],
        out_specs=pl.BlockSpec((1, 28, 28, 192), lambda i: (i, 0, 0, 0)),
        out_shape=jax.ShapeDtypeStruct((N, 28, 28, 192), _BF),
        compiler_params=_cparams(),
    )(s1, w2, conv2_bias, w3s, conv3_bias)

    v = _inception(s2, "i3a", i3a_w1, i3a_b1, i3a_w3, i3a_b3, "none")
    v = _inception(v, "i3b", i3b_w1, i3b_b1, i3b_w3, i3b_b3, "pool3")
    v = _inception(v, "i4a", i4a_w1, i4a_b1, i4a_w3, i4a_b3, "none")
    v = _inception(v, "i4b", i4b_w1, i4b_b1, i4b_w3, i4b_b3, "none")
    v = _inception(v, "i4c", i4c_w1, i4c_b1, i4c_w3, i4c_b3, "none")
    v = _inception(v, "i4d", i4d_w1, i4d_b1, i4d_w3, i4d_b3, "none")
    v = _inception(v, "i4e", i4e_w1, i4e_b1, i4e_w3, i4e_b3, "pool2")
    v = _inception(v, "i5a", i5a_w1, i5a_b1, i5a_w3, i5a_b3, "none")
    feats = _inception(v, "i5b", i5b_w1, i5b_b1, i5b_w3, i5b_b3, "mean")

    out = pl.pallas_call(
        _head_kernel,
        out_shape=jax.ShapeDtypeStruct((N, head_w.shape[1]), _F32),
    )(feats, fc_w.astype(_BF), fc_b, head_w.astype(_BF), head_b)
    return out[:, :10]


# speedup vs baseline: 3.6337x; 2.2526x over previous
import functools

import numpy as np

import jax
import jax.numpy as jnp
from jax import lax
from jax.experimental import pallas as pl
from jax.experimental.pallas import tpu as pltpu

_BF = jnp.bfloat16
_F32 = jnp.float32
_VMEM = 64 * 1024 * 1024

_INC = {
    "i3a": (192, 64, 96, 128, 16, 32, 32, 2),
    "i3b": (256, 128, 128, 192, 32, 96, 64, 2),
    "i4a": (480, 192, 96, 208, 16, 48, 64, 4),
    "i4b": (512, 160, 112, 224, 24, 64, 64, 4),
    "i4c": (512, 128, 128, 256, 24, 64, 64, 4),
    "i4d": (512, 112, 144, 288, 32, 64, 64, 4),
    "i4e": (528, 256, 160, 320, 32, 128, 128, 4),
    "i5a": (832, 256, 160, 320, 32, 128, 128, 8),
    "i5b": (832, 384, 192, 384, 48, 128, 128, 8),
}


def _cparams():
    return pltpu.CompilerParams(dimension_semantics=("parallel",),
                                vmem_limit_bytes=_VMEM)


def _full_spec(a):
    n = a.ndim
    return pl.BlockSpec(a.shape, lambda i, _n=n: (0,) * _n)


def _pad_hw(v, top, bottom, left, right):
    nb, H, W, C = v.shape
    dt = v.dtype
    if left or right:
        pieces = []
        if left:
            pieces.append(jnp.zeros((nb, H, left, C), dt))
        pieces.append(v)
        if right:
            pieces.append(jnp.zeros((nb, H, right, C), dt))
        v = jnp.concatenate(pieces, axis=2)
    if top or bottom:
        W2 = v.shape[2]
        pieces = []
        if top:
            pieces.append(jnp.zeros((nb, top, W2, C), dt))
        pieces.append(v)
        if bottom:
            pieces.append(jnp.zeros((nb, bottom, W2, C), dt))
        v = jnp.concatenate(pieces, axis=1)
    return v


def _maxpool3s1(v):
    nb, H, W, C = v.shape
    p = _pad_hw(v, 1, 1, 1, 1)
    out = None
    for di in range(3):
        for dj in range(3):
            s = p[:, di:di + H, dj:dj + W, :]
            out = s if out is None else jnp.maximum(out, s)
    return out


def _maxpool3s2(v):
    nb, H, W, C = v.shape
    Ho, Wo = H // 2, W // 2
    p = _pad_hw(v, 0, 2, 0, 0)
    r = p.reshape(nb, Ho + 1, 2, W, C)
    a = jnp.maximum(jnp.maximum(r[:, :Ho, 0], r[:, :Ho, 1]), r[:, 1:Ho + 1, 0])
    p2 = _pad_hw(a, 0, 0, 0, 2)
    c = p2.reshape(nb, Ho, Wo + 1, 2, C)
    return jnp.maximum(jnp.maximum(c[:, :, :Wo, 0], c[:, :, :Wo, 1]),
                       c[:, :, 1:Wo + 1, 0])


def _maxpool2s2(v):
    nb, H, W, C = v.shape
    Ho, Wo = H // 2, W // 2
    r = v.reshape(nb, Ho, 2, W, C)
    a = jnp.maximum(r[:, :, 0], r[:, :, 1])
    c = a.reshape(nb, Ho, Wo, 2, C)
    return jnp.maximum(c[:, :, :, 0], c[:, :, :, 1])


def _conv3x3(vpad, w3_ref, H, W):
    nb = vpad.shape[0]
    Cr = vpad.shape[-1]
    acc = None
    for di in range(3):
        cat = jnp.concatenate(
            [vpad[:, di:di + H, dj:dj + W, :] for dj in range(3)], axis=-1)
        d = jnp.dot(cat.reshape(nb * H * W, 3 * Cr), w3_ref[di],
                    preferred_element_type=_F32)
        acc = d if acc is None else acc + d
    return acc


def _s2d_kernel(x_ref, e_ref, o_ref):
    x = x_ref[0]
    row = lax.broadcasted_iota(jnp.int32, x.shape, 0)
    x = x + jnp.where(row == 0, -0.03, jnp.where(row == 1, -0.088, -0.188))
    t = lax.dot_general(x.astype(_BF), e_ref[...], (((0,), (0,)), ((), ())),
                        preferred_element_type=_F32)
    v = t.astype(_BF).reshape(28, 2, 112, 2, 128)
    x12 = jnp.concatenate(
        [v[:, p, :, q, :3] for p in range(2) for q in range(2)], axis=-1)
    o_ref[...] = x12.reshape(1, 28, 112, 12)


def _stem1_kernel(x_ref, w_ref, b_ref, o_ref):
    xpp = _pad_hw(x_ref[...], 2, 1, 2, 1)
    xcat = jnp.concatenate(
        [xpp[:, :, rb:rb + 112, :] for rb in range(4)], axis=-1)
    acc = None
    for ra in range(4):
        d = jnp.dot(xcat[0, ra:ra + 112].reshape(12544, 48), w_ref[ra],
                    preferred_element_type=_F32)
        acc = d if acc is None else acc + d
    a = jnp.maximum(acc + b_ref[...], 0.0)
    v = a.astype(_BF).reshape(1, 112, 112, 64)
    o_ref[...] = _maxpool3s2(v)


def _stem2_kernel(x_ref, w2_ref, b2_ref, w3_ref, b3_ref, o_ref):
    x = x_ref[...]
    y2 = jnp.dot(x.reshape(3136, 64), w2_ref[...], preferred_element_type=_F32)
    y2 = jnp.maximum(y2 + b2_ref[...], 0.0).astype(_BF).reshape(1, 56, 56, 64)
    yp = _pad_hw(y2, 1, 1, 1, 1)
    y3 = _conv3x3(yp, w3_ref, 56, 56)
    y3 = jnp.maximum(y3 + b3_ref[...], 0.0).astype(_BF).reshape(1, 56, 56, 192)
    o_ref[...] = _maxpool3s2(y3)


def _inc_kernel(dims, post, x_ref, w1m_ref, b1m_ref, w1p_ref, b1p_ref,
                w3_ref, b3_ref, o_ref):
    c1, c3, c5, cp, Cr = dims
    nb, H, W, C = x_ref.shape
    M = nb * H * W
    x = x_ref[...]
    pooled = _maxpool3s1(x)
    ymain = jnp.dot(x.reshape(M, C), w1m_ref[...], preferred_element_type=_F32)
    ymain = jnp.maximum(ymain + b1m_ref[...], 0.0)
    b4 = jnp.dot(pooled.reshape(M, C), w1p_ref[...],
                 preferred_element_type=_F32)
    b4 = jnp.maximum(b4 + b1p_ref[...], 0.0)
    red = ymain[:, c1:].astype(_BF).reshape(nb, H, W, Cr)
    y3 = _conv3x3(_pad_hw(red, 1, 1, 1, 1), w3_ref, H, W)
    y3 = jnp.maximum(y3 + b3_ref[...], 0.0)
    parts = ((0, c1, ymain[:, :c1]), (c1, c3 + c5, y3),
             (c1 + c3 + c5, cp, b4))
    if post == "mean":
        for col, wdt, v in parts:
            o_ref[:, col:col + wdt] = jnp.mean(
                v.reshape(nb, H * W, wdt), axis=1)
    else:
        for col, wdt, v in parts:
            vb = v.astype(_BF).reshape(nb, H, W, wdt)
            if post == "pool3":
                vb = _maxpool3s2(vb)
            elif post == "pool2":
                vb = _maxpool2s2(vb)
            o_ref[:, :, :, col:col + wdt] = vb


def _head_kernel(f_ref, fcw_ref, fcb_ref, hw_ref, hb_ref, o_ref):
    t = jnp.dot(f_ref[...].astype(_BF), fcw_ref[...],
                preferred_element_type=_F32)
    t = jnp.maximum(t + fcb_ref[...], 0.0)
    logits = jnp.dot(t.astype(_BF), hw_ref[...],
                     preferred_element_type=_F32) + hb_ref[...]
    col = lax.broadcasted_iota(jnp.int32, logits.shape, 1)
    mask = col < 10
    logits = jnp.where(mask, logits, -jnp.inf)
    m = jnp.max(logits, axis=-1, keepdims=True)
    e = jnp.where(mask, jnp.exp(logits - m), 0.0)
    o_ref[...] = e / jnp.sum(e, axis=-1, keepdims=True)


def _rearrange_w3(w3, Cr, N):
    return (w3.reshape(Cr, 3, 3, N).transpose(1, 2, 0, 3)
            .reshape(3, 3 * Cr, N).astype(_BF))


def _inception(x, name, w1, b1, w3, b3, post):
    cin, c1, c3r, c3, c5r, c5, cp, nb = _INC[name]
    nmain = c1 + c3r + c5r
    Cr = c3r + c5r
    N, H, W, C = x.shape
    w1m = w1[:cin, :nmain].astype(_BF)
    w1p = w1[cin:, nmain:].astype(_BF)
    b1m = b1[:, :nmain]
    b1p = b1[:, nmain:]
    w3r = _rearrange_w3(w3, Cr, c3 + c5)
    cout = c1 + c3 + c5 + cp
    if post == "mean":
        out_shape = jax.ShapeDtypeStruct((N, cout), _F32)
        out_spec = pl.BlockSpec((nb, cout), lambda i: (i, 0))
    elif post in ("pool3", "pool2"):
        out_shape = jax.ShapeDtypeStruct((N, H // 2, W // 2, cout), _BF)
        out_spec = pl.BlockSpec((nb, H // 2, W // 2, cout),
                                lambda i: (i, 0, 0, 0))
    else:
        out_shape = jax.ShapeDtypeStruct((N, H, W, cout), _BF)
        out_spec = pl.BlockSpec((nb, H, W, cout), lambda i: (i, 0, 0, 0))
    kern = functools.partial(_inc_kernel, (c1, c3, c5, cp, Cr), post)
    return pl.pallas_call(
        kern,
        grid=(N // nb,),
        in_specs=[pl.BlockSpec((nb, H, W, C), lambda i: (i, 0, 0, 0)),
                  _full_spec(w1m), _full_spec(b1m), _full_spec(w1p),
                  _full_spec(b1p), _full_spec(w3r), _full_spec(b3)],
        out_specs=out_spec,
        out_shape=out_shape,
        compiler_params=_cparams(),
    )(x, w1m, b1m, w1p, b1p, w3r, b3)


def kernel(x, conv1_wmat, conv1_bias, conv2_wmat, conv2_bias, conv3_wmat,
           conv3_bias, i3a_w1, i3a_b1, i3a_w3, i3a_b3, i3b_w1, i3b_b1,
           i3b_w3, i3b_b3, i4a_w1, i4a_b1, i4a_w3, i4a_b3, i4b_w1, i4b_b1,
           i4b_w3, i4b_b3, i4c_w1, i4c_b1, i4c_w3, i4c_b3, i4d_w1, i4d_b1,
           i4d_w3, i4d_b3, i4e_w1, i4e_b1, i4e_w3, i4e_b3, i5a_w1, i5a_b1,
           i5a_w3, i5a_b3, i5b_w1, i5b_b1, i5b_w3, i5b_b3, fc_w, fc_b,
           head_w, head_b):
    N = x.shape[0]
    xflat = x.reshape(N, 3, 50176)
    eye3 = jnp.eye(3, 128, dtype=_BF)

    idx = np.full((4, 4, 2, 2, 3), 147, np.int32)
    for ra in range(4):
        for rb in range(4):
            for p in range(2):
                for q in range(2):
                    i, j = 2 * ra + p - 1, 2 * rb + q - 1
                    if 0 <= i < 7 and 0 <= j < 7:
                        for c in range(3):
                            idx[ra, rb, p, q, c] = c * 49 + i * 7 + j
    scale = jnp.array([0.229 / 0.5, 0.224 / 0.5, 0.225 / 0.5], _F32)
    w1s = (conv1_wmat.reshape(3, 49, 64) * scale[:, None, None]).reshape(147, 64)
    w_ext = jnp.concatenate([w1s, jnp.zeros((1, 64), _F32)], axis=0)
    w1 = jnp.take(w_ext, jnp.asarray(idx.reshape(4, 48)), axis=0).astype(_BF)

    x12 = pl.pallas_call(
        _s2d_kernel,
        grid=(N, 4),
        in_specs=[pl.BlockSpec((1, 3, 12544), lambda i, h: (i, 0, h)),
                  pl.BlockSpec((3, 128), lambda i, h: (0, 0))],
        out_specs=pl.BlockSpec((1, 28, 112, 12), lambda i, h: (i, h, 0, 0)),
        out_shape=jax.ShapeDtypeStruct((N, 112, 112, 12), _BF),
        compiler_params=pltpu.CompilerParams(
            dimension_semantics=("parallel", "arbitrary"),
            vmem_limit_bytes=_VMEM),
    )(xflat, eye3)

    s1 = pl.pallas_call(
        _stem1_kernel,
        grid=(N,),
        in_specs=[pl.BlockSpec((1, 112, 112, 12), lambda i: (i, 0, 0, 0)),
                  _full_spec(w1), _full_spec(conv1_bias)],
        out_specs=pl.BlockSpec((1, 56, 56, 64), lambda i: (i, 0, 0, 0)),
        out_shape=jax.ShapeDtypeStruct((N, 56, 56, 64), _BF),
        compiler_params=_cparams(),
    )(x12, w1, conv1_bias)

    w2 = conv2_wmat.astype(_BF)
    w3s = _rearrange_w3(conv3_wmat, 64, 192)
    s2 = pl.pallas_call(
        _stem2_kernel,
        grid=(N,),
        in_specs=[pl.BlockSpec((1, 56, 56, 64), lambda i: (i, 0, 0, 0)),
                  _full_spec(w2), _full_spec(conv2_bias),
                  _full_spec(w3s), _full_spec(conv3_bias)],
        out_specs=pl.BlockSpec((1, 28, 28, 192), lambda i: (i, 0, 0, 0)),
        out_shape=jax.ShapeDtypeStruct((N, 28, 28, 192), _BF),
        compiler_params=_cparams(),
    )(s1, w2, conv2_bias, w3s, conv3_bias)

    v = _inception(s2, "i3a", i3a_w1, i3a_b1, i3a_w3, i3a_b3, "none")
    v = _inception(v, "i3b", i3b_w1, i3b_b1, i3b_w3, i3b_b3, "pool3")
    v = _inception(v, "i4a", i4a_w1, i4a_b1, i4a_w3, i4a_b3, "none")
    v = _inception(v, "i4b", i4b_w1, i4b_b1, i4b_w3, i4b_b3, "none")
    v = _inception(v, "i4c", i4c_w1, i4c_b1, i4c_w3, i4c_b3, "none")
    v = _inception(v, "i4d", i4d_w1, i4d_b1, i4d_w3, i4d_b3, "none")
    v = _inception(v, "i4e", i4e_w1, i4e_b1, i4e_w3, i4e_b3, "pool2")
    v = _inception(v, "i5a", i5a_w1, i5a_b1, i5a_w3, i5a_b3, "none")
    feats = _inception(v, "i5b", i5b_w1, i5b_b1, i5b_w3, i5b_b3, "mean")

    out = pl.pallas_call(
        _head_kernel,
        out_shape=jax.ShapeDtypeStruct((N, head_w.shape[1]), _F32),
    )(feats, fc_w.astype(_BF), fc_b, head_w.astype(_BF), head_b)
    return out[:, :10]


# speedup vs baseline: 5.8874x; 1.6202x over previous
import functools

import numpy as np

import jax
import jax.numpy as jnp
from jax import lax
from jax.experimental import pallas as pl
from jax.experimental.pallas import tpu as pltpu

_BF = jnp.bfloat16
_F32 = jnp.float32
_VMEM = 64 * 1024 * 1024

_INC = {
    "i3a": (192, 64, 96, 128, 16, 32, 32, 2),
    "i3b": (256, 128, 128, 192, 32, 96, 64, 2),
    "i4a": (480, 192, 96, 208, 16, 48, 64, 4),
    "i4b": (512, 160, 112, 224, 24, 64, 64, 4),
    "i4c": (512, 128, 128, 256, 24, 64, 64, 4),
    "i4d": (512, 112, 144, 288, 32, 64, 64, 4),
    "i4e": (528, 256, 160, 320, 32, 128, 128, 4),
    "i5a": (832, 256, 160, 320, 32, 128, 128, 8),
    "i5b": (832, 384, 192, 384, 48, 128, 128, 8),
}


def _cparams():
    return pltpu.CompilerParams(dimension_semantics=("parallel",),
                                vmem_limit_bytes=_VMEM)


def _full_spec(a):
    n = a.ndim
    return pl.BlockSpec(a.shape, lambda i, _n=n: (0,) * _n)


def _pad_hw(v, top, bottom, left, right):
    nb, H, W, C = v.shape
    dt = v.dtype
    if left or right:
        pieces = []
        if left:
            pieces.append(jnp.zeros((nb, H, left, C), dt))
        pieces.append(v)
        if right:
            pieces.append(jnp.zeros((nb, H, right, C), dt))
        v = jnp.concatenate(pieces, axis=2)
    if top or bottom:
        W2 = v.shape[2]
        pieces = []
        if top:
            pieces.append(jnp.zeros((nb, top, W2, C), dt))
        pieces.append(v)
        if bottom:
            pieces.append(jnp.zeros((nb, bottom, W2, C), dt))
        v = jnp.concatenate(pieces, axis=1)
    return v


def _maxpool3s1(v):
    nb, H, W, C = v.shape
    p = _pad_hw(v, 1, 1, 1, 1)
    out = None
    for di in range(3):
        for dj in range(3):
            s = p[:, di:di + H, dj:dj + W, :]
            out = s if out is None else jnp.maximum(out, s)
    return out


def _maxpool3s2(v):
    nb, H, W, C = v.shape
    Ho, Wo = H // 2, W // 2
    p = _pad_hw(v, 0, 2, 0, 0)
    r = p.reshape(nb, Ho + 1, 2, W, C)
    a = jnp.maximum(jnp.maximum(r[:, :Ho, 0], r[:, :Ho, 1]), r[:, 1:Ho + 1, 0])
    p2 = _pad_hw(a, 0, 0, 0, 2)
    c = p2.reshape(nb, Ho, Wo + 1, 2, C)
    return jnp.maximum(jnp.maximum(c[:, :, :Wo, 0], c[:, :, :Wo, 1]),
                       c[:, :, 1:Wo + 1, 0])


def _maxpool2s2(v):
    nb, H, W, C = v.shape
    Ho, Wo = H // 2, W // 2
    r = v.reshape(nb, Ho, 2, W, C)
    a = jnp.maximum(r[:, :, 0], r[:, :, 1])
    c = a.reshape(nb, Ho, Wo, 2, C)
    return jnp.maximum(c[:, :, :, 0], c[:, :, :, 1])


def _conv3x3(vpad, w3_ref, H, W):
    nb = vpad.shape[0]
    Cr = vpad.shape[-1]
    acc = None
    for di in range(3):
        cat = jnp.concatenate(
            [vpad[:, di:di + H, dj:dj + W, :] for dj in range(3)], axis=-1)
        d = jnp.dot(cat.reshape(nb * H * W, 3 * Cr), w3_ref[di],
                    preferred_element_type=_F32)
        acc = d if acc is None else acc + d
    return acc


def _s2d_kernel(x_ref, e_ref, o_ref):
    x = x_ref[0]
    row = lax.broadcasted_iota(jnp.int32, x.shape, 0)
    x = x + jnp.where(row == 0, -0.03, jnp.where(row == 1, -0.088, -0.188))
    t = lax.dot_general(x.astype(_BF), e_ref[...], (((0,), (0,)), ((), ())),
                        preferred_element_type=_F32)
    tw = pltpu.bitcast(t.astype(_BF), jnp.int32)
    we = pltpu.unpack_elementwise(
        tw, index=0, packed_dtype=_BF, unpacked_dtype=_F32).reshape(
            28, 2, 112, 128)
    wo = pltpu.unpack_elementwise(
        tw, index=1, packed_dtype=_BF, unpacked_dtype=_F32).reshape(
            28, 2, 112, 128)
    x12 = jnp.concatenate(
        [we[:, 0, :, :3], wo[:, 0, :, :3], we[:, 1, :, :3], wo[:, 1, :, :3]],
        axis=-1)
    o_ref[...] = x12.astype(_BF).reshape(1, 28, 112, 12)


def _stem1_kernel(x_ref, w_ref, b_ref, o_ref):
    xpp = _pad_hw(x_ref[...], 2, 1, 2, 1)
    xcat = jnp.concatenate(
        [xpp[:, :, rb:rb + 112, :] for rb in range(4)], axis=-1)
    acc = None
    for ra in range(4):
        d = jnp.dot(xcat[0, ra:ra + 112].reshape(12544, 48), w_ref[ra],
                    preferred_element_type=_F32)
        acc = d if acc is None else acc + d
    a = jnp.maximum(acc + b_ref[...], 0.0)
    v = a.astype(_BF).reshape(1, 112, 112, 64)
    o_ref[...] = _maxpool3s2(v)


def _stem2_kernel(x_ref, w2_ref, b2_ref, w3_ref, b3_ref, o_ref):
    x = x_ref[...]
    y2 = jnp.dot(x.reshape(3136, 64), w2_ref[...], preferred_element_type=_F32)
    y2 = jnp.maximum(y2 + b2_ref[...], 0.0).astype(_BF).reshape(1, 56, 56, 64)
    yp = _pad_hw(y2, 1, 1, 1, 1)
    y3 = _conv3x3(yp, w3_ref, 56, 56)
    y3 = jnp.maximum(y3 + b3_ref[...], 0.0).astype(_BF).reshape(1, 56, 56, 192)
    o_ref[...] = _maxpool3s2(y3)


def _inc_kernel(dims, post, x_ref, w1m_ref, b1m_ref, w1p_ref, b1p_ref,
                w3_ref, b3_ref, o_ref):
    c1, c3, c5, cp, Cr = dims
    nb, H, W, C = x_ref.shape
    M = nb * H * W
    x = x_ref[...]
    pooled = _maxpool3s1(x)
    ymain = jnp.dot(x.reshape(M, C), w1m_ref[...], preferred_element_type=_F32)
    ymain = jnp.maximum(ymain + b1m_ref[...], 0.0)
    b4 = jnp.dot(pooled.reshape(M, C), w1p_ref[...],
                 preferred_element_type=_F32)
    b4 = jnp.maximum(b4 + b1p_ref[...], 0.0)
    red = ymain[:, c1:].astype(_BF).reshape(nb, H, W, Cr)
    y3 = _conv3x3(_pad_hw(red, 1, 1, 1, 1), w3_ref, H, W)
    y3 = jnp.maximum(y3 + b3_ref[...], 0.0)
    parts = ((0, c1, ymain[:, :c1]), (c1, c3 + c5, y3),
             (c1 + c3 + c5, cp, b4))
    if post == "mean":
        for col, wdt, v in parts:
            o_ref[:, col:col + wdt] = jnp.mean(
                v.reshape(nb, H * W, wdt), axis=1)
    else:
        for col, wdt, v in parts:
            vb = v.astype(_BF).reshape(nb, H, W, wdt)
            if post == "pool3":
                vb = _maxpool3s2(vb)
            elif post == "pool2":
                vb = _maxpool2s2(vb)
            o_ref[:, :, :, col:col + wdt] = vb


def _head_kernel(f_ref, fcw_ref, fcb_ref, hw_ref, hb_ref, o_ref):
    t = jnp.dot(f_ref[...].astype(_BF), fcw_ref[...],
                preferred_element_type=_F32)
    t = jnp.maximum(t + fcb_ref[...], 0.0)
    logits = jnp.dot(t.astype(_BF), hw_ref[...],
                     preferred_element_type=_F32) + hb_ref[...]
    col = lax.broadcasted_iota(jnp.int32, logits.shape, 1)
    mask = col < 10
    logits = jnp.where(mask, logits, -jnp.inf)
    m = jnp.max(logits, axis=-1, keepdims=True)
    e = jnp.where(mask, jnp.exp(logits - m), 0.0)
    o_ref[...] = e / jnp.sum(e, axis=-1, keepdims=True)


def _rearrange_w3(w3, Cr, N):
    return (w3.reshape(Cr, 3, 3, N).transpose(1, 2, 0, 3)
            .reshape(3, 3 * Cr, N).astype(_BF))


def _inception(x, name, w1, b1, w3, b3, post):
    cin, c1, c3r, c3, c5r, c5, cp, nb = _INC[name]
    nmain = c1 + c3r + c5r
    Cr = c3r + c5r
    N, H, W, C = x.shape
    w1m = w1[:cin, :nmain].astype(_BF)
    w1p = w1[cin:, nmain:].astype(_BF)
    b1m = b1[:, :nmain]
    b1p = b1[:, nmain:]
    w3r = _rearrange_w3(w3, Cr, c3 + c5)
    cout = c1 + c3 + c5 + cp
    if post == "mean":
        out_shape = jax.ShapeDtypeStruct((N, cout), _F32)
        out_spec = pl.BlockSpec((nb, cout), lambda i: (i, 0))
    elif post in ("pool3", "pool2"):
        out_shape = jax.ShapeDtypeStruct((N, H // 2, W // 2, cout), _BF)
        out_spec = pl.BlockSpec((nb, H // 2, W // 2, cout),
                                lambda i: (i, 0, 0, 0))
    else:
        out_shape = jax.ShapeDtypeStruct((N, H, W, cout), _BF)
        out_spec = pl.BlockSpec((nb, H, W, cout), lambda i: (i, 0, 0, 0))
    kern = functools.partial(_inc_kernel, (c1, c3, c5, cp, Cr), post)
    return pl.pallas_call(
        kern,
        grid=(N // nb,),
        in_specs=[pl.BlockSpec((nb, H, W, C), lambda i: (i, 0, 0, 0)),
                  _full_spec(w1m), _full_spec(b1m), _full_spec(w1p),
                  _full_spec(b1p), _full_spec(w3r), _full_spec(b3)],
        out_specs=out_spec,
        out_shape=out_shape,
        compiler_params=_cparams(),
    )(x, w1m, b1m, w1p, b1p, w3r, b3)


def kernel(x, conv1_wmat, conv1_bias, conv2_wmat, conv2_bias, conv3_wmat,
           conv3_bias, i3a_w1, i3a_b1, i3a_w3, i3a_b3, i3b_w1, i3b_b1,
           i3b_w3, i3b_b3, i4a_w1, i4a_b1, i4a_w3, i4a_b3, i4b_w1, i4b_b1,
           i4b_w3, i4b_b3, i4c_w1, i4c_b1, i4c_w3, i4c_b3, i4d_w1, i4d_b1,
           i4d_w3, i4d_b3, i4e_w1, i4e_b1, i4e_w3, i4e_b3, i5a_w1, i5a_b1,
           i5a_w3, i5a_b3, i5b_w1, i5b_b1, i5b_w3, i5b_b3, fc_w, fc_b,
           head_w, head_b):
    N = x.shape[0]
    xflat = x.reshape(N, 3, 50176)
    eye3 = jnp.eye(3, 128, dtype=_BF)

    idx = np.full((4, 4, 2, 2, 3), 147, np.int32)
    for ra in range(4):
        for rb in range(4):
            for p in range(2):
                for q in range(2):
                    i, j = 2 * ra + p - 1, 2 * rb + q - 1
                    if 0 <= i < 7 and 0 <= j < 7:
                        for c in range(3):
                            idx[ra, rb, p, q, c] = c * 49 + i * 7 + j
    scale = jnp.array([0.229 / 0.5, 0.224 / 0.5, 0.225 / 0.5], _F32)
    w1s = (conv1_wmat.reshape(3, 49, 64) * scale[:, None, None]).reshape(147, 64)
    w_ext = jnp.concatenate([w1s, jnp.zeros((1, 64), _F32)], axis=0)
    w1 = jnp.take(w_ext, jnp.asarray(idx.reshape(4, 48)), axis=0).astype(_BF)

    x12 = pl.pallas_call(
        _s2d_kernel,
        grid=(N, 4),
        in_specs=[pl.BlockSpec((1, 3, 12544), lambda i, h: (i, 0, h)),
                  pl.BlockSpec((3, 128), lambda i, h: (0, 0))],
        out_specs=pl.BlockSpec((1, 28, 112, 12), lambda i, h: (i, h, 0, 0)),
        out_shape=jax.ShapeDtypeStruct((N, 112, 112, 12), _BF),
        compiler_params=pltpu.CompilerParams(
            dimension_semantics=("parallel", "arbitrary"),
            vmem_limit_bytes=_VMEM),
    )(xflat, eye3)

    s1 = pl.pallas_call(
        _stem1_kernel,
        grid=(N,),
        in_specs=[pl.BlockSpec((1, 112, 112, 12), lambda i: (i, 0, 0, 0)),
                  _full_spec(w1), _full_spec(conv1_bias)],
        out_specs=pl.BlockSpec((1, 56, 56, 64), lambda i: (i, 0, 0, 0)),
        out_shape=jax.ShapeDtypeStruct((N, 56, 56, 64), _BF),
        compiler_params=_cparams(),
    )(x12, w1, conv1_bias)

    w2 = conv2_wmat.astype(_BF)
    w3s = _rearrange_w3(conv3_wmat, 64, 192)
    s2 = pl.pallas_call(
        _stem2_kernel,
        grid=(N,),
        in_specs=[pl.BlockSpec((1, 56, 56, 64), lambda i: (i, 0, 0, 0)),
                  _full_spec(w2), _full_spec(conv2_bias),
                  _full_spec(w3s), _full_spec(conv3_bias)],
        out_specs=pl.BlockSpec((1, 28, 28, 192), lambda i: (i, 0, 0, 0)),
        out_shape=jax.ShapeDtypeStruct((N, 28, 28, 192), _BF),
        compiler_params=_cparams(),
    )(s1, w2, conv2_bias, w3s, conv3_bias)

    v = _inception(s2, "i3a", i3a_w1, i3a_b1, i3a_w3, i3a_b3, "none")
    v = _inception(v, "i3b", i3b_w1, i3b_b1, i3b_w3, i3b_b3, "pool3")
    v = _inception(v, "i4a", i4a_w1, i4a_b1, i4a_w3, i4a_b3, "none")
    v = _inception(v, "i4b", i4b_w1, i4b_b1, i4b_w3, i4b_b3, "none")
    v = _inception(v, "i4c", i4c_w1, i4c_b1, i4c_w3, i4c_b3, "none")
    v = _inception(v, "i4d", i4d_w1, i4d_b1, i4d_w3, i4d_b3, "none")
    v = _inception(v, "i4e", i4e_w1, i4e_b1, i4e_w3, i4e_b3, "pool2")
    v = _inception(v, "i5a", i5a_w1, i5a_b1, i5a_w3, i5a_b3, "none")
    feats = _inception(v, "i5b", i5b_w1, i5b_b1, i5b_w3, i5b_b3, "mean")

    out = pl.pallas_call(
        _head_kernel,
        out_shape=jax.ShapeDtypeStruct((N, head_w.shape[1]), _F32),
    )(feats, fc_w.astype(_BF), fc_b, head_w.astype(_BF), head_b)
    return out[:, :10]


# speedup vs baseline: 6.8320x; 1.1604x over previous
import functools

import numpy as np

import jax
import jax.numpy as jnp
from jax import lax
from jax.experimental import pallas as pl
from jax.experimental.pallas import tpu as pltpu

_BF = jnp.bfloat16
_F32 = jnp.float32
_VMEM = 64 * 1024 * 1024

_INC = {
    "i3a": (192, 64, 96, 128, 16, 32, 32, 2),
    "i3b": (256, 128, 128, 192, 32, 96, 64, 2),
    "i4a": (480, 192, 96, 208, 16, 48, 64, 4),
    "i4b": (512, 160, 112, 224, 24, 64, 64, 4),
    "i4c": (512, 128, 128, 256, 24, 64, 64, 4),
    "i4d": (512, 112, 144, 288, 32, 64, 64, 4),
    "i4e": (528, 256, 160, 320, 32, 128, 128, 4),
    "i5a": (832, 256, 160, 320, 32, 128, 128, 8),
    "i5b": (832, 384, 192, 384, 48, 128, 128, 8),
}


def _cparams():
    return pltpu.CompilerParams(dimension_semantics=("parallel",),
                                vmem_limit_bytes=_VMEM)


def _full_spec(a):
    n = a.ndim
    return pl.BlockSpec(a.shape, lambda i, _n=n: (0,) * _n)


def _pad_hw(v, top, bottom, left, right):
    nb, H, W, C = v.shape
    dt = v.dtype
    if left or right:
        pieces = []
        if left:
            pieces.append(jnp.zeros((nb, H, left, C), dt))
        pieces.append(v)
        if right:
            pieces.append(jnp.zeros((nb, H, right, C), dt))
        v = jnp.concatenate(pieces, axis=2)
    if top or bottom:
        W2 = v.shape[2]
        pieces = []
        if top:
            pieces.append(jnp.zeros((nb, top, W2, C), dt))
        pieces.append(v)
        if bottom:
            pieces.append(jnp.zeros((nb, bottom, W2, C), dt))
        v = jnp.concatenate(pieces, axis=1)
    return v


def _maxpool3s1(v):
    nb, H, W, C = v.shape
    p = _pad_hw(v, 1, 1, 1, 1)
    out = None
    for di in range(3):
        for dj in range(3):
            s = p[:, di:di + H, dj:dj + W, :]
            out = s if out is None else jnp.maximum(out, s)
    return out


def _maxpool3s2(v):
    nb, H, W, C = v.shape
    Ho, Wo = H // 2, W // 2
    p = _pad_hw(v, 0, 2, 0, 0)
    r = p.reshape(nb, Ho + 1, 2, W, C)
    a = jnp.maximum(jnp.maximum(r[:, :Ho, 0], r[:, :Ho, 1]), r[:, 1:Ho + 1, 0])
    p2 = _pad_hw(a, 0, 0, 0, 2)
    c = p2.reshape(nb, Ho, Wo + 1, 2, C)
    return jnp.maximum(jnp.maximum(c[:, :, :Wo, 0], c[:, :, :Wo, 1]),
                       c[:, :, 1:Wo + 1, 0])


def _maxpool2s2(v):
    nb, H, W, C = v.shape
    Ho, Wo = H // 2, W // 2
    r = v.reshape(nb, Ho, 2, W, C)
    a = jnp.maximum(r[:, :, 0], r[:, :, 1])
    c = a.reshape(nb, Ho, Wo, 2, C)
    return jnp.maximum(c[:, :, :, 0], c[:, :, :, 1])


def _wpairs(x):
    pr = pltpu.bitcast(x, jnp.int32)
    e = pltpu.unpack_elementwise(pr, index=0, packed_dtype=_BF,
                                 unpacked_dtype=_F32)
    o = pltpu.unpack_elementwise(pr, index=1, packed_dtype=_BF,
                                 unpacked_dtype=_F32)
    return e, o


def _maxpool3s2_2d(x, nb, H, W):
    C = x.shape[1]
    z1 = jnp.zeros((W, C), _BF)
    Wo = W // 2

    def one(xi):
        s1 = jnp.concatenate([xi[W:], z1], axis=0)
        s2 = jnp.concatenate([xi[2 * W:], z1, z1], axis=0)
        mh = jnp.maximum(jnp.maximum(xi, s1), s2)
        he = mh.reshape(H // 2, 2 * W, C)[:, :W, :].reshape((H // 2) * W, C)
        e, o = _wpairs(he)
        m1 = jnp.maximum(e, o)
        esh = jnp.concatenate([e[1:], jnp.zeros((1, C), _F32)], axis=0)
        wo = lax.broadcasted_iota(jnp.int32, esh.shape, 0) % Wo
        esh = jnp.where(wo == Wo - 1, 0.0, esh)
        return jnp.maximum(m1, esh).astype(_BF)

    return jnp.concatenate(
        [one(x[i * H * W:(i + 1) * H * W]) for i in range(nb)], axis=0)


def _maxpool2s2_2d(x, nb, H, W):
    C = x.shape[1]
    z1 = jnp.zeros((W, C), _BF)

    def one(xi):
        mh = jnp.maximum(xi, jnp.concatenate([xi[W:], z1], axis=0))
        he = mh.reshape(H // 2, 2 * W, C)[:, :W, :].reshape((H // 2) * W, C)
        e, o = _wpairs(he)
        return jnp.maximum(e, o).astype(_BF)

    return jnp.concatenate(
        [one(x[i * H * W:(i + 1) * H * W]) for i in range(nb)], axis=0)


def _conv3x3(vpad, w3_ref, H, W):
    nb = vpad.shape[0]
    Cr = vpad.shape[-1]
    acc = None
    for di in range(3):
        cat = jnp.concatenate(
            [vpad[:, di:di + H, dj:dj + W, :] for dj in range(3)], axis=-1)
        d = jnp.dot(cat.reshape(nb * H * W, 3 * Cr), w3_ref[di],
                    preferred_element_type=_F32)
        acc = d if acc is None else acc + d
    return acc


def _s2d_kernel(x_ref, e_ref, o_ref):
    x = x_ref[0]
    row = lax.broadcasted_iota(jnp.int32, x.shape, 0)
    x = x + jnp.where(row == 0, -0.03, jnp.where(row == 1, -0.088, -0.188))
    t = lax.dot_general(x.astype(_BF), e_ref[...], (((0,), (0,)), ((), ())),
                        preferred_element_type=_F32)
    tw = pltpu.bitcast(t.astype(_BF), jnp.int32)
    we = pltpu.unpack_elementwise(
        tw, index=0, packed_dtype=_BF, unpacked_dtype=_F32).reshape(
            28, 2, 112, 128)
    wo = pltpu.unpack_elementwise(
        tw, index=1, packed_dtype=_BF, unpacked_dtype=_F32).reshape(
            28, 2, 112, 128)
    x12 = jnp.concatenate(
        [we[:, 0, :, :3], wo[:, 0, :, :3], we[:, 1, :, :3], wo[:, 1, :, :3]],
        axis=-1)
    o_ref[...] = x12.astype(_BF).reshape(1, 28, 112, 12)


def _stem1_kernel(x_ref, w_ref, b_ref, o_ref):
    xpp = _pad_hw(x_ref[...], 2, 1, 2, 1)
    xcat = jnp.concatenate(
        [xpp[:, :, rb:rb + 112, :] for rb in range(4)], axis=-1)
    acc = None
    for ra in range(4):
        d = jnp.dot(xcat[0, ra:ra + 112].reshape(12544, 48), w_ref[ra],
                    preferred_element_type=_F32)
        acc = d if acc is None else acc + d
    a = jnp.maximum(acc + b_ref[...], 0.0)
    p = _maxpool3s2_2d(a.astype(_BF), 1, 112, 112)
    o_ref[...] = p.reshape(1, 56, 56, 64)


def _stem2_kernel(x_ref, w2_ref, b2_ref, w3_ref, b3_ref, o_ref):
    x = x_ref[...]
    y2 = jnp.dot(x.reshape(3136, 64), w2_ref[...], preferred_element_type=_F32)
    y2 = jnp.maximum(y2 + b2_ref[...], 0.0).astype(_BF).reshape(1, 56, 56, 64)
    yp = _pad_hw(y2, 1, 1, 1, 1)
    y3 = _conv3x3(yp, w3_ref, 56, 56)
    y3 = jnp.maximum(y3 + b3_ref[...], 0.0).astype(_BF)
    o_ref[...] = _maxpool3s2_2d(y3, 1, 56, 56).reshape(1, 28, 28, 192)


def _inc_kernel(dims, post, x_ref, w1m_ref, b1m_ref, w1p_ref, b1p_ref,
                w3_ref, b3_ref, o_ref):
    c1, c3, c5, cp, Cr = dims
    nb, H, W, C = x_ref.shape
    M = nb * H * W
    x = x_ref[...]
    pooled = _maxpool3s1(x)
    ymain = jnp.dot(x.reshape(M, C), w1m_ref[...], preferred_element_type=_F32)
    ymain = jnp.maximum(ymain + b1m_ref[...], 0.0)
    b4 = jnp.dot(pooled.reshape(M, C), w1p_ref[...],
                 preferred_element_type=_F32)
    b4 = jnp.maximum(b4 + b1p_ref[...], 0.0)
    red = ymain[:, c1:].astype(_BF).reshape(nb, H, W, Cr)
    y3 = _conv3x3(_pad_hw(red, 1, 1, 1, 1), w3_ref, H, W)
    y3 = jnp.maximum(y3 + b3_ref[...], 0.0)
    parts = ((0, c1, ymain[:, :c1]), (c1, c3 + c5, y3),
             (c1 + c3 + c5, cp, b4))
    if post == "mean":
        for col, wdt, v in parts:
            o_ref[:, col:col + wdt] = jnp.mean(
                v.reshape(nb, H * W, wdt), axis=1)
    else:
        for col, wdt, v in parts:
            vb = v.astype(_BF)
            if post == "pool3":
                vb = _maxpool3s2_2d(vb, nb, H, W).reshape(
                    nb, H // 2, W // 2, wdt)
            elif post == "pool2":
                vb = _maxpool2s2_2d(vb, nb, H, W).reshape(
                    nb, H // 2, W // 2, wdt)
            else:
                vb = vb.reshape(nb, H, W, wdt)
            o_ref[:, :, :, col:col + wdt] = vb


def _head_kernel(f_ref, fcw_ref, fcb_ref, hw_ref, hb_ref, o_ref):
    t = jnp.dot(f_ref[...].astype(_BF), fcw_ref[...],
                preferred_element_type=_F32)
    t = jnp.maximum(t + fcb_ref[...], 0.0)
    logits = jnp.dot(t.astype(_BF), hw_ref[...],
                     preferred_element_type=_F32) + hb_ref[...]
    col = lax.broadcasted_iota(jnp.int32, logits.shape, 1)
    mask = col < 10
    logits = jnp.where(mask, logits, -jnp.inf)
    m = jnp.max(logits, axis=-1, keepdims=True)
    e = jnp.where(mask, jnp.exp(logits - m), 0.0)
    o_ref[...] = e / jnp.sum(e, axis=-1, keepdims=True)


def _rearrange_w3(w3, Cr, N):
    return (w3.reshape(Cr, 3, 3, N).transpose(1, 2, 0, 3)
            .reshape(3, 3 * Cr, N).astype(_BF))


def _inception(x, name, w1, b1, w3, b3, post):
    cin, c1, c3r, c3, c5r, c5, cp, nb = _INC[name]
    nmain = c1 + c3r + c5r
    Cr = c3r + c5r
    N, H, W, C = x.shape
    w1m = w1[:cin, :nmain].astype(_BF)
    w1p = w1[cin:, nmain:].astype(_BF)
    b1m = b1[:, :nmain]
    b1p = b1[:, nmain:]
    w3r = _rearrange_w3(w3, Cr, c3 + c5)
    cout = c1 + c3 + c5 + cp
    if post == "mean":
        out_shape = jax.ShapeDtypeStruct((N, cout), _F32)
        out_spec = pl.BlockSpec((nb, cout), lambda i: (i, 0))
    elif post in ("pool3", "pool2"):
        out_shape = jax.ShapeDtypeStruct((N, H // 2, W // 2, cout), _BF)
        out_spec = pl.BlockSpec((nb, H // 2, W // 2, cout),
                                lambda i: (i, 0, 0, 0))
    else:
        out_shape = jax.ShapeDtypeStruct((N, H, W, cout), _BF)
        out_spec = pl.BlockSpec((nb, H, W, cout), lambda i: (i, 0, 0, 0))
    kern = functools.partial(_inc_kernel, (c1, c3, c5, cp, Cr), post)
    return pl.pallas_call(
        kern,
        grid=(N // nb,),
        in_specs=[pl.BlockSpec((nb, H, W, C), lambda i: (i, 0, 0, 0)),
                  _full_spec(w1m), _full_spec(b1m), _full_spec(w1p),
                  _full_spec(b1p), _full_spec(w3r), _full_spec(b3)],
        out_specs=out_spec,
        out_shape=out_shape,
        compiler_params=_cparams(),
    )(x, w1m, b1m, w1p, b1p, w3r, b3)


def kernel(x, conv1_wmat, conv1_bias, conv2_wmat, conv2_bias, conv3_wmat,
           conv3_bias, i3a_w1, i3a_b1, i3a_w3, i3a_b3, i3b_w1, i3b_b1,
           i3b_w3, i3b_b3, i4a_w1, i4a_b1, i4a_w3, i4a_b3, i4b_w1, i4b_b1,
           i4b_w3, i4b_b3, i4c_w1, i4c_b1, i4c_w3, i4c_b3, i4d_w1, i4d_b1,
           i4d_w3, i4d_b3, i4e_w1, i4e_b1, i4e_w3, i4e_b3, i5a_w1, i5a_b1,
           i5a_w3, i5a_b3, i5b_w1, i5b_b1, i5b_w3, i5b_b3, fc_w, fc_b,
           head_w, head_b):
    N = x.shape[0]
    xflat = x.reshape(N, 3, 50176)
    eye3 = jnp.eye(3, 128, dtype=_BF)

    idx = np.full((4, 4, 2, 2, 3), 147, np.int32)
    for ra in range(4):
        for rb in range(4):
            for p in range(2):
                for q in range(2):
                    i, j = 2 * ra + p - 1, 2 * rb + q - 1
                    if 0 <= i < 7 and 0 <= j < 7:
                        for c in range(3):
                            idx[ra, rb, p, q, c] = c * 49 + i * 7 + j
    scale = jnp.array([0.229 / 0.5, 0.224 / 0.5, 0.225 / 0.5], _F32)
    w1s = (conv1_wmat.reshape(3, 49, 64) * scale[:, None, None]).reshape(147, 64)
    w_ext = jnp.concatenate([w1s, jnp.zeros((1, 64), _F32)], axis=0)
    w1 = jnp.take(w_ext, jnp.asarray(idx.reshape(4, 48)), axis=0).astype(_BF)

    x12 = pl.pallas_call(
        _s2d_kernel,
        grid=(N, 4),
        in_specs=[pl.BlockSpec((1, 3, 12544), lambda i, h: (i, 0, h)),
                  pl.BlockSpec((3, 128), lambda i, h: (0, 0))],
        out_specs=pl.BlockSpec((1, 28, 112, 12), lambda i, h: (i, h, 0, 0)),
        out_shape=jax.ShapeDtypeStruct((N, 112, 112, 12), _BF),
        compiler_params=pltpu.CompilerParams(
            dimension_semantics=("parallel", "arbitrary"),
            vmem_limit_bytes=_VMEM),
    )(xflat, eye3)

    s1 = pl.pallas_call(
        _stem1_kernel,
        grid=(N,),
        in_specs=[pl.BlockSpec((1, 112, 112, 12), lambda i: (i, 0, 0, 0)),
                  _full_spec(w1), _full_spec(conv1_bias)],
        out_specs=pl.BlockSpec((1, 56, 56, 64), lambda i: (i, 0, 0, 0)),
        out_shape=jax.ShapeDtypeStruct((N, 56, 56, 64), _BF),
        compiler_params=_cparams(),
    )(x12, w1, conv1_bias)

    w2 = conv2_wmat.astype(_BF)
    w3s = _rearrange_w3(conv3_wmat, 64, 192)
    s2 = pl.pallas_call(
        _stem2_kernel,
        grid=(N,),
        in_specs=[pl.BlockSpec((1, 56, 56, 64), lambda i: (i, 0, 0, 0)),
                  _full_spec(w2), _full_spec(conv2_bias),
                  _full_spec(w3s), _full_spec(conv3_bias)],
        out_specs=pl.BlockSpec((1, 28, 28, 192), lambda i: (i, 0, 0, 0)),
        out_shape=jax.ShapeDtypeStruct((N, 28, 28, 192), _BF),
        compiler_params=_cparams(),
    )(s1, w2, conv2_bias, w3s, conv3_bias)

    v = _inception(s2, "i3a", i3a_w1, i3a_b1, i3a_w3, i3a_b3, "none")
    v = _inception(v, "i3b", i3b_w1, i3b_b1, i3b_w3, i3b_b3, "pool3")
    v = _inception(v, "i4a", i4a_w1, i4a_b1, i4a_w3, i4a_b3, "none")
    v = _inception(v, "i4b", i4b_w1, i4b_b1, i4b_w3, i4b_b3, "none")
    v = _inception(v, "i4c", i4c_w1, i4c_b1, i4c_w3, i4c_b3, "none")
    v = _inception(v, "i4d", i4d_w1, i4d_b1, i4d_w3, i4d_b3, "none")
    v = _inception(v, "i4e", i4e_w1, i4e_b1, i4e_w3, i4e_b3, "pool2")
    v = _inception(v, "i5a", i5a_w1, i5a_b1, i5a_w3, i5a_b3, "none")
    feats = _inception(v, "i5b", i5b_w1, i5b_b1, i5b_w3, i5b_b3, "mean")

    out = pl.pallas_call(
        _head_kernel,
        out_shape=jax.ShapeDtypeStruct((N, head_w.shape[1]), _F32),
    )(feats, fc_w.astype(_BF), fc_b, head_w.astype(_BF), head_b)
    return out[:, :10]
